# baseline (speedup 1.0000x reference)
import functools

import jax
import jax.numpy as jnp
from jax import lax
from jax.experimental import pallas as pl
from jax.experimental.pallas import tpu as pltpu
from jax.experimental.pallas import tpu_sc as plsc

B, N, D, P = 64, 677, 768, 100
NPATCH = N - (1 + P)
MAP_SIZE = 1000
ACC = 128
NL = 16
NCOL = D // NL
CK = 64
NCHUNKS = NPATCH // CK


def _tc_q_body(q_ref, vecs_ref, qvpt_ref, qacc):
    b = pl.program_id(0)
    inv_np = jnp.float32(1.0 / NPATCH)
    inv_b = jnp.float32(1.0 / B)
    tok = lax.broadcasted_iota(jnp.int32, (1, N), 1)
    wp = jnp.where(tok >= 1 + P, inv_np, 0.0).astype(jnp.float32)
    dn = (((1,), (0,)), ((), ()))
    qrow = q_ref[0]
    vecs_ref[0, 0:1, :] = lax.dot_general(wp, qrow, dn,
                                          preferred_element_type=jnp.float32)
    vecs_ref[0, 1:2, :] = qrow[0:1, :]
    qv = qrow[0:ACC, :] * inv_b

    @pl.when(b == 0)
    def _():
        qacc[...] = qv

    @pl.when(b != 0)
    def _():
        qacc[...] += qv

    @pl.when(b == B - 1)
    def _():
        qvpt_ref[...] = qacc[1:1 + P, :]


_tc_q_call = pl.pallas_call(
    _tc_q_body,
    grid=(B,),
    in_specs=[pl.BlockSpec((1, N, D), lambda b: (b, 0, 0))],
    out_specs=[pl.BlockSpec((1, 2, D), lambda b: (b, 0, 0)),
               pl.BlockSpec((P, D), lambda b: (0, 0))],
    out_shape=[jax.ShapeDtypeStruct((B, 2, D), jnp.float32),
               jax.ShapeDtypeStruct((P, D), jnp.float32)],
    scratch_shapes=[pltpu.VMEM((ACC, D), jnp.float32)],
)


PSTART = 96
PSKIP = (1 + P) - PSTART
NROWS = N - PSTART
TAIL = NROWS - NCHUNKS * CK
NWIN = 13
WROWS = 8


@functools.lru_cache(maxsize=None)
def _sc_reduce_fn():
    mesh = plsc.VectorSubcoreMesh(core_axis_name="c", subcore_axis_name="s")

    @functools.partial(
        pl.kernel,
        mesh=mesh,
        compiler_params=pltpu.CompilerParams(needs_layout_passes=False),
        out_type=[
            jax.ShapeDtypeStruct((32, 2, D), jnp.float32),
            jax.ShapeDtypeStruct((32, 2, D), jnp.float32),
            jax.ShapeDtypeStruct((32, 2, D), jnp.float32),
            jax.ShapeDtypeStruct((32, 2, D), jnp.float32),
            jax.ShapeDtypeStruct((NWIN, WROWS, D), jnp.float32),
        ],
        scratch_types=[
            pltpu.VMEM((CK, D), jnp.float32),
            pltpu.VMEM((CK, D), jnp.float32),
            pltpu.VMEM((8, D), jnp.float32),
            pltpu.VMEM((2, D), jnp.float32),
            pltpu.VMEM((2, D), jnp.float32),
            pltpu.VMEM((WROWS, D), jnp.float32),
            pltpu.SemaphoreType.DMA,
            pltpu.SemaphoreType.DMA,
            pltpu.SemaphoreType.DMA,
            pltpu.SemaphoreType.DMA,
        ],
    )
    def _sc_reduce(batch_hbm, k_hbm,
                   bpatch_hbm, bcls_hbm, kpatch_hbm, kcls_hbm, kvpt_hbm,
                   buf0, buf1, clsw, prow, crow, vacc,
                   sem0, sem1, sem_cls, sem_out):
        wid = lax.axis_index("s") * 2 + lax.axis_index("c")
        bufs = (buf0, buf1)
        sems = (sem0, sem1)
        inv_np = jnp.float32(1.0 / NPATCH)
        inv_b = jnp.float32(1.0 / B)

        def chunk_copy(arr_hbm, s, ci, buf, sem, rows):
            return pltpu.make_async_copy(
                arr_hbm.at[s, pl.ds(PSTART + ci * CK, rows), :],
                buf.at[pl.ds(0, rows), :] if rows != CK else buf, sem)

        G = 8
        NG = NCOL // G

        def accum(buf, i, lo, rows):
            for g in range(NG):
                def body(r, carry, buf=buf, g=g):
                    return tuple(
                        carry[u] + buf[r, pl.ds((g * G + u) * NL, NL)]
                        for u in range(G))

                acc = tuple(
                    prow[i, pl.ds((g * G + u) * NL, NL)] for u in range(G))
                acc = lax.fori_loop(lo, rows, body, acc)
                for u in range(G):
                    prow[i, pl.ds((g * G + u) * NL, NL)] = acc[u]

        def patch_stream(arr_hbm, i):
            s = wid * 2 + i
            pltpu.make_async_copy(
                arr_hbm.at[s, pl.ds(0, 8), :], clsw, sem_cls).start()
            chunk_copy(arr_hbm, s, 0, buf0, sem0, CK).start()
            chunk_copy(arr_hbm, s, 1, buf1, sem1, CK).start()
            zero = jnp.zeros((NL,), jnp.float32)
            for c in range(NCOL):
                prow[i, pl.ds(c * NL, NL)] = zero

            def pair_body(pi, _):
                c0 = 2 * pi
                chunk_copy(arr_hbm, s, c0, buf0, sem0, CK).wait()
                accum(buf0, i, jnp.where(pi == 0, PSKIP, 0), CK)
                chunk_copy(arr_hbm, s, c0 + 2, buf0, sem0, CK).start()
                chunk_copy(arr_hbm, s, c0 + 1, buf1, sem1, CK).wait()
                accum(buf1, i, 0, CK)

                @pl.when(pi < (NCHUNKS - 3) // 2)
                def _():
                    chunk_copy(arr_hbm, s, c0 + 3, buf1, sem1, CK).start()
                return 0

            lax.fori_loop(0, (NCHUNKS - 1) // 2, pair_body, 0)
            chunk_copy(arr_hbm, s, NCHUNKS - 1, buf0, sem0, CK).wait()
            accum(buf0, i, 0, CK)
            for c in range(NCOL):
                prow[i, pl.ds(c * NL, NL)] = (
                    prow[i, pl.ds(c * NL, NL)] * inv_np)
            pltpu.make_async_copy(
                arr_hbm.at[s, pl.ds(0, 8), :], clsw, sem_cls).wait()
            for c in range(NCOL):
                crow[i, pl.ds(c * NL, NL)] = clsw[0, pl.ds(c * NL, NL)]

        for arr_hbm, pout, cout in ((batch_hbm, bpatch_hbm, bcls_hbm),
                                    (k_hbm, kpatch_hbm, kcls_hbm)):
            def sample_body(i, _, arr_hbm=arr_hbm):
                patch_stream(arr_hbm, i)
                return 0

            lax.fori_loop(0, 2, sample_body, 0)
            pltpu.sync_copy(prow, pout.at[wid])
            pltpu.sync_copy(crow, cout.at[wid])

        @pl.when(wid < NWIN)
        def _():
            for c in range(NCOL):
                zero = jnp.zeros((NL,), jnp.float32)
                for tt in range(WROWS):
                    vacc[tt, pl.ds(c * NL, NL)] = zero

            def sub_pass(sub, _):
                def start_one(j, _):
                    pltpu.make_async_copy(
                        k_hbm.at[sub * 8 + j, pl.ds(wid * WROWS, WROWS), :],
                        buf0.at[pl.ds(j * WROWS, WROWS), :], sem0).start()
                    return 0

                lax.fori_loop(0, 8, start_one, 0)
                pltpu.make_async_copy(
                    k_hbm.at[0, pl.ds(0, CK), :], buf0, sem0).wait()

                def tok_body(tt, _):
                    for g in range(NG):
                        def row_body(j, carry, g=g):
                            return tuple(
                                carry[u] + buf0[j * WROWS + tt,
                                                pl.ds((g * G + u) * NL, NL)]
                                for u in range(G))

                        acc = tuple(
                            vacc[tt, pl.ds((g * G + u) * NL, NL)]
                            for u in range(G))
                        acc = lax.fori_loop(0, 8, row_body, acc)
                        for u in range(G):
                            vacc[tt, pl.ds((g * G + u) * NL, NL)] = acc[u]
                    return 0

                lax.fori_loop(0, WROWS, tok_body, 0)
                return 0

            lax.fori_loop(0, 8, sub_pass, 0)
            for c in range(NCOL):
                for tt in range(WROWS):
                    vacc[tt, pl.ds(c * NL, NL)] = (
                        vacc[tt, pl.ds(c * NL, NL)] * inv_b)
            pltpu.sync_copy(vacc, kvpt_hbm.at[wid])

    return _sc_reduce


@functools.lru_cache(maxsize=None)
def _sc_map_fn():
    mesh = plsc.VectorSubcoreMesh(core_axis_name="c", subcore_axis_name="s")

    @functools.partial(
        pl.kernel,
        mesh=mesh,
        compiler_params=pltpu.CompilerParams(needs_layout_passes=False),
        out_type=[jax.ShapeDtypeStruct((B,), jnp.int32)],
        scratch_types=[
            pltpu.VMEM((B,), jnp.int32),
            pltpu.VMEM((MAP_SIZE,), jnp.int32),
            pltpu.VMEM((B,), jnp.int32),
        ],
    )
    def _sc_map(labels_hbm, mapping_hbm, mapped_hbm,
                labels_v, mapping_v, mapped_v):
        wid = lax.axis_index("s") * 2 + lax.axis_index("c")

        @pl.when(wid == 0)
        def _():
            pltpu.sync_copy(labels_hbm, labels_v)
            pltpu.sync_copy(mapping_hbm, mapping_v)
            for i in range(B // NL):
                lab = labels_v[pl.ds(i * NL, NL)]
                mapped_v[pl.ds(i * NL, NL)] = plsc.load_gather(mapping_v, [lab])
            pltpu.sync_copy(mapped_v, mapped_hbm)

    return _sc_map


def _gather_body(m_ref, out_hbm, b_hbm, k_hbm,
                 op_ref, ov_ref, bt_ref, kt_ref, tb, tk, sem):
    copies = []
    for i in range(B):
        r = 1 + m_ref[i]
        copies.append(pltpu.make_async_copy(
            out_hbm.at[i, pl.ds(r, 1), :], op_ref.at[i], sem))
        copies.append(pltpu.make_async_copy(
            out_hbm.at[i, pl.ds(0, 1), :], ov_ref.at[i], sem))
        copies.append(pltpu.make_async_copy(
            b_hbm.at[i, pl.ds(N - TAIL, TAIL), :], tb.at[i], sem))
        copies.append(pltpu.make_async_copy(
            k_hbm.at[i, pl.ds(N - TAIL, TAIL), :], tk.at[i], sem))
    for c in copies:
        c.start()
    for c in copies:
        c.wait()
    inv_np = jnp.float32(1.0 / NPATCH)
    bt_ref[...] = jnp.sum(tb[...], axis=1, keepdims=True) * inv_np
    kt_ref[...] = jnp.sum(tk[...], axis=1, keepdims=True) * inv_np


_gather_call = pl.pallas_call(
    _gather_body,
    grid_spec=pltpu.PrefetchScalarGridSpec(
        num_scalar_prefetch=1,
        grid=(1,),
        in_specs=[pl.BlockSpec(memory_space=pl.ANY)] * 3,
        out_specs=[pl.BlockSpec(memory_space=pltpu.MemorySpace.VMEM)] * 4,
        scratch_shapes=[pltpu.VMEM((B, TAIL, D), jnp.float32),
                        pltpu.VMEM((B, TAIL, D), jnp.float32),
                        pltpu.SemaphoreType.DMA],
    ),
    out_shape=[jax.ShapeDtypeStruct((B, 1, D), jnp.float32)] * 4,
)


def kernel(batch, vpt, q, k, labels, output, mapping):
    qvecs, qvpt = _tc_q_call(q)
    bpatch3, bcls3, kpatch3, kcls3, kvpt_raw = _sc_reduce_fn()(batch, k)
    (mapped,) = _sc_map_fn()(labels, mapping)
    out_patch3, out_vpt3, btail3, ktail3 = _gather_call(
        mapped, output, batch, k)
    bpatch = bpatch3.reshape(B, D) + btail3[:, 0]
    bcls = bcls3.reshape(B, D)
    kpatch = kpatch3.reshape(B, D) + ktail3[:, 0]
    kcls = kcls3.reshape(B, D)
    kvpt = kvpt_raw.reshape(NWIN * WROWS, D)[1:1 + P]
    return (bpatch, qvecs[:, 0], kpatch, out_patch3[:, 0], vpt,
            qvpt[None], kvpt[None], out_vpt3[:, 0][None],
            bcls, qvecs[:, 1], kcls, mapped)


# speedup vs baseline: 1.0560x; 1.0560x over previous
import functools

import jax
import jax.numpy as jnp
from jax import lax
from jax.experimental import pallas as pl
from jax.experimental.pallas import tpu as pltpu
from jax.experimental.pallas import tpu_sc as plsc

B, N, D, P = 64, 677, 768, 100
NPATCH = N - (1 + P)
MAP_SIZE = 1000
ACC = 128
NL = 16
NCOL = D // NL
CK = 64
NCHUNKS = NPATCH // CK


def _tc_q_body(q_ref, vecs_ref, qvpt_ref, qacc):
    b = pl.program_id(0)
    inv_np = jnp.float32(1.0 / NPATCH)
    inv_b = jnp.float32(1.0 / B)
    tok = lax.broadcasted_iota(jnp.int32, (1, N), 1)
    wp = jnp.where(tok >= 1 + P, inv_np, 0.0).astype(jnp.float32)
    dn = (((1,), (0,)), ((), ()))
    qrow = q_ref[0]
    vecs_ref[0, 0:1, :] = lax.dot_general(wp, qrow, dn,
                                          preferred_element_type=jnp.float32)
    vecs_ref[0, 1:2, :] = qrow[0:1, :]
    qv = qrow[0:ACC, :] * inv_b

    @pl.when(b == 0)
    def _():
        qacc[...] = qv

    @pl.when(b != 0)
    def _():
        qacc[...] += qv

    @pl.when(b == B - 1)
    def _():
        qvpt_ref[...] = qacc[1:1 + P, :]


_tc_q_call = pl.pallas_call(
    _tc_q_body,
    grid=(B,),
    in_specs=[pl.BlockSpec((1, N, D), lambda b: (b, 0, 0))],
    out_specs=[pl.BlockSpec((1, 2, D), lambda b: (b, 0, 0)),
               pl.BlockSpec((P, D), lambda b: (0, 0))],
    out_shape=[jax.ShapeDtypeStruct((B, 2, D), jnp.float32),
               jax.ShapeDtypeStruct((P, D), jnp.float32)],
    scratch_shapes=[pltpu.VMEM((ACC, D), jnp.float32)],
)


PSTART = 96
PSKIP = (1 + P) - PSTART
NROWS = N - PSTART
TAIL = NROWS - NCHUNKS * CK
NWIN = 13
WROWS = 8


@functools.lru_cache(maxsize=None)
def _sc_reduce_fn():
    mesh = plsc.VectorSubcoreMesh(core_axis_name="c", subcore_axis_name="s")

    @functools.partial(
        pl.kernel,
        mesh=mesh,
        compiler_params=pltpu.CompilerParams(needs_layout_passes=False),
        out_type=[
            jax.ShapeDtypeStruct((32, 2, D), jnp.float32),
            jax.ShapeDtypeStruct((32, 2, D), jnp.float32),
            jax.ShapeDtypeStruct((32, 2, D), jnp.float32),
            jax.ShapeDtypeStruct((32, 2, D), jnp.float32),
            jax.ShapeDtypeStruct((NWIN, WROWS, D), jnp.float32),
        ],
        scratch_types=[
            pltpu.VMEM((CK, D), jnp.float32),
            pltpu.VMEM((CK, D), jnp.float32),
            pltpu.VMEM((8, D), jnp.float32),
            pltpu.VMEM((2, D), jnp.float32),
            pltpu.VMEM((2, D), jnp.float32),
            pltpu.VMEM((WROWS, D), jnp.float32),
            pltpu.SemaphoreType.DMA,
            pltpu.SemaphoreType.DMA,
            pltpu.SemaphoreType.DMA,
            pltpu.SemaphoreType.DMA,
        ],
    )
    def _sc_reduce(batch_hbm, k_hbm,
                   bpatch_hbm, bcls_hbm, kpatch_hbm, kcls_hbm, kvpt_hbm,
                   buf0, buf1, clsw, prow, crow, vacc,
                   sem0, sem1, sem_cls, sem_out):
        wid = lax.axis_index("s") * 2 + lax.axis_index("c")
        bufs = (buf0, buf1)
        sems = (sem0, sem1)
        inv_np = jnp.float32(1.0 / NPATCH)
        inv_b = jnp.float32(1.0 / B)

        def chunk_copy(arr_hbm, s, ci, buf, sem, rows):
            return pltpu.make_async_copy(
                arr_hbm.at[s, pl.ds(PSTART + ci * CK, rows), :],
                buf.at[pl.ds(0, rows), :] if rows != CK else buf, sem)

        G = 8
        NG = NCOL // G

        def accum(buf, i, rows, sign=1.0, unroll=4):
            sgn = jnp.float32(sign)
            for g in range(NG):
                def body(r, carry, buf=buf, g=g):
                    return tuple(
                        carry[u] + sgn * buf[r, pl.ds((g * G + u) * NL, NL)]
                        for u in range(G))

                acc = tuple(
                    prow[i, pl.ds((g * G + u) * NL, NL)] for u in range(G))
                acc = lax.fori_loop(0, rows, body, acc, unroll=unroll)
                for u in range(G):
                    prow[i, pl.ds((g * G + u) * NL, NL)] = acc[u]

        def patch_stream(arr_hbm, i):
            s = wid * 2 + i
            pltpu.make_async_copy(
                arr_hbm.at[s, pl.ds(0, 8), :], clsw, sem_cls).start()
            pltpu.make_async_copy(
                arr_hbm.at[s, pl.ds(PSTART, 8), :], vacc, sem_out).start()
            chunk_copy(arr_hbm, s, 0, buf0, sem0, CK).start()
            chunk_copy(arr_hbm, s, 1, buf1, sem1, CK).start()
            zero = jnp.zeros((NL,), jnp.float32)
            for c in range(NCOL):
                prow[i, pl.ds(c * NL, NL)] = zero

            def pair_body(pi, _):
                c0 = 2 * pi
                chunk_copy(arr_hbm, s, c0, buf0, sem0, CK).wait()
                accum(buf0, i, CK)
                chunk_copy(arr_hbm, s, c0 + 2, buf0, sem0, CK).start()
                chunk_copy(arr_hbm, s, c0 + 1, buf1, sem1, CK).wait()
                accum(buf1, i, CK)

                @pl.when(pi < (NCHUNKS - 3) // 2)
                def _():
                    chunk_copy(arr_hbm, s, c0 + 3, buf1, sem1, CK).start()
                return 0

            lax.fori_loop(0, (NCHUNKS - 1) // 2, pair_body, 0)
            chunk_copy(arr_hbm, s, NCHUNKS - 1, buf0, sem0, CK).wait()
            accum(buf0, i, CK)
            pltpu.make_async_copy(
                arr_hbm.at[s, pl.ds(PSTART, 8), :], vacc, sem_out).wait()
            accum(vacc, i, PSKIP, sign=-1.0, unroll=PSKIP)
            for c in range(NCOL):
                prow[i, pl.ds(c * NL, NL)] = (
                    prow[i, pl.ds(c * NL, NL)] * inv_np)
            pltpu.make_async_copy(
                arr_hbm.at[s, pl.ds(0, 8), :], clsw, sem_cls).wait()
            for c in range(NCOL):
                crow[i, pl.ds(c * NL, NL)] = clsw[0, pl.ds(c * NL, NL)]

        for arr_hbm, pout, cout in ((batch_hbm, bpatch_hbm, bcls_hbm),
                                    (k_hbm, kpatch_hbm, kcls_hbm)):
            def sample_body(i, _, arr_hbm=arr_hbm):
                patch_stream(arr_hbm, i)
                return 0

            lax.fori_loop(0, 2, sample_body, 0)
            pltpu.sync_copy(prow, pout.at[wid])
            pltpu.sync_copy(crow, cout.at[wid])

        @pl.when(wid < NWIN)
        def _():
            for c in range(NCOL):
                zero = jnp.zeros((NL,), jnp.float32)
                for tt in range(WROWS):
                    vacc[tt, pl.ds(c * NL, NL)] = zero

            def sub_pass(sub, _):
                def start_one(j, _):
                    pltpu.make_async_copy(
                        k_hbm.at[sub * 8 + j, pl.ds(wid * WROWS, WROWS), :],
                        buf0.at[pl.ds(j * WROWS, WROWS), :], sem0).start()
                    return 0

                lax.fori_loop(0, 8, start_one, 0)
                pltpu.make_async_copy(
                    k_hbm.at[0, pl.ds(0, CK), :], buf0, sem0).wait()

                def tok_body(tt, _):
                    for g in range(NG):
                        def row_body(j, carry, g=g):
                            return tuple(
                                carry[u] + buf0[j * WROWS + tt,
                                                pl.ds((g * G + u) * NL, NL)]
                                for u in range(G))

                        acc = tuple(
                            vacc[tt, pl.ds((g * G + u) * NL, NL)]
                            for u in range(G))
                        acc = lax.fori_loop(0, 8, row_body, acc, unroll=8)
                        for u in range(G):
                            vacc[tt, pl.ds((g * G + u) * NL, NL)] = acc[u]
                    return 0

                lax.fori_loop(0, WROWS, tok_body, 0)
                return 0

            lax.fori_loop(0, 8, sub_pass, 0)
            for c in range(NCOL):
                for tt in range(WROWS):
                    vacc[tt, pl.ds(c * NL, NL)] = (
                        vacc[tt, pl.ds(c * NL, NL)] * inv_b)
            pltpu.sync_copy(vacc, kvpt_hbm.at[wid])

    return _sc_reduce


@functools.lru_cache(maxsize=None)
def _sc_map_fn():
    mesh = plsc.VectorSubcoreMesh(core_axis_name="c", subcore_axis_name="s")

    @functools.partial(
        pl.kernel,
        mesh=mesh,
        compiler_params=pltpu.CompilerParams(needs_layout_passes=False),
        out_type=[jax.ShapeDtypeStruct((B,), jnp.int32)],
        scratch_types=[
            pltpu.VMEM((B,), jnp.int32),
            pltpu.VMEM((MAP_SIZE,), jnp.int32),
            pltpu.VMEM((B,), jnp.int32),
        ],
    )
    def _sc_map(labels_hbm, mapping_hbm, mapped_hbm,
                labels_v, mapping_v, mapped_v):
        wid = lax.axis_index("s") * 2 + lax.axis_index("c")

        @pl.when(wid == 0)
        def _():
            pltpu.sync_copy(labels_hbm, labels_v)
            pltpu.sync_copy(mapping_hbm, mapping_v)
            for i in range(B // NL):
                lab = labels_v[pl.ds(i * NL, NL)]
                mapped_v[pl.ds(i * NL, NL)] = plsc.load_gather(mapping_v, [lab])
            pltpu.sync_copy(mapped_v, mapped_hbm)

    return _sc_map


def _gather_body(m_ref, out_hbm, b_hbm, k_hbm,
                 op_ref, ov_ref, bt_ref, kt_ref, tb, tk, sem):
    copies = []
    for i in range(B):
        r = 1 + m_ref[i]
        copies.append(pltpu.make_async_copy(
            out_hbm.at[i, pl.ds(r, 1), :], op_ref.at[i], sem))
        copies.append(pltpu.make_async_copy(
            out_hbm.at[i, pl.ds(0, 1), :], ov_ref.at[i], sem))
        copies.append(pltpu.make_async_copy(
            b_hbm.at[i, pl.ds(N - TAIL, TAIL), :], tb.at[i], sem))
        copies.append(pltpu.make_async_copy(
            k_hbm.at[i, pl.ds(N - TAIL, TAIL), :], tk.at[i], sem))
    for c in copies:
        c.start()
    for c in copies:
        c.wait()
    inv_np = jnp.float32(1.0 / NPATCH)
    bt_ref[...] = jnp.sum(tb[...], axis=1, keepdims=True) * inv_np
    kt_ref[...] = jnp.sum(tk[...], axis=1, keepdims=True) * inv_np


_gather_call = pl.pallas_call(
    _gather_body,
    grid_spec=pltpu.PrefetchScalarGridSpec(
        num_scalar_prefetch=1,
        grid=(1,),
        in_specs=[pl.BlockSpec(memory_space=pl.ANY)] * 3,
        out_specs=[pl.BlockSpec(memory_space=pltpu.MemorySpace.VMEM)] * 4,
        scratch_shapes=[pltpu.VMEM((B, TAIL, D), jnp.float32),
                        pltpu.VMEM((B, TAIL, D), jnp.float32),
                        pltpu.SemaphoreType.DMA],
    ),
    out_shape=[jax.ShapeDtypeStruct((B, 1, D), jnp.float32)] * 4,
)


def kernel(batch, vpt, q, k, labels, output, mapping):
    bpatch3, bcls3, kpatch3, kcls3, kvpt_raw = _sc_reduce_fn()(batch, k)
    (mapped,) = _sc_map_fn()(labels, mapping)
    qvecs, qvpt = _tc_q_call(q)
    out_patch3, out_vpt3, btail3, ktail3 = _gather_call(
        mapped, output, batch, k)
    bpatch = bpatch3.reshape(B, D) + btail3[:, 0]
    bcls = bcls3.reshape(B, D)
    kpatch = kpatch3.reshape(B, D) + ktail3[:, 0]
    kcls = kcls3.reshape(B, D)
    kvpt = kvpt_raw.reshape(NWIN * WROWS, D)[1:1 + P]
    return (bpatch, qvecs[:, 0], kpatch, out_patch3[:, 0], vpt,
            qvpt[None], kvpt[None], out_vpt3[:, 0][None],
            bcls, qvecs[:, 1], kcls, mapped)
